# baseline (speedup 1.0000x reference)
import jax
import jax.numpy as jnp
from jax.experimental import pallas as pl
from jax.experimental.pallas import tpu as pltpu

_D = 256
_H = 256
_BLK = 512
_TILE = 128
_HALF = 4096


def _gru(h, gi, gh):
    r = jax.nn.sigmoid(gi[:, :_H] + gh[:, :_H])
    z = jax.nn.sigmoid(gi[:, _H:2 * _H] + gh[:, _H:2 * _H])
    n = jnp.tanh(gi[:, 2 * _H:] + r * gh[:, 2 * _H:])
    return (1.0 - z) * n + z * h


def _pool_scale(adj):
    smean = jnp.mean(adj, axis=1, keepdims=True)
    smax = jnp.max(adj, axis=1, keepdims=True)
    return 1.0 + smean + smax


def _gcn_body(f1_ref, adj1_ref, f2_ref, adj2_ref, w1_ref, b1_ref, wg_ref,
              wih_ref, whh_ref, bih_ref, bhh_ref, out1_ref, out2_ref):
    pid = pl.program_id(0)

    def indep_tile(f_ref, adj_ref, out_ref, t):
        rs = pl.ds(t * _TILE, _TILE)
        f = f_ref[rs, :]
        xw = jnp.dot(f, w1_ref[:], preferred_element_type=jnp.float32)
        x = jnp.maximum(xw + b1_ref[:], 0.0)
        x = jnp.maximum(x * _pool_scale(adj_ref[rs, :]), 0.0)
        bsum = bih_ref[:] + bhh_ref[:]
        h = x
        for _ in range(2):
            g = jnp.dot(h, whh_ref[:], preferred_element_type=jnp.float32)
            r = jax.nn.sigmoid(g[:, :_H] + bsum[:, :_H])
            z = jax.nn.sigmoid(g[:, _H:2 * _H] + bsum[:, _H:2 * _H])
            hn = g[:, 2 * _H:] + bhh_ref[:, 2 * _H:]
            n = jnp.tanh(bih_ref[:, 2 * _H:] + r * hn)
            h = (1.0 - z) * n + z * h
        out_ref[rs, :] = jnp.maximum(h, 0.0)

    @pl.when(pid != 0)
    def _():
        for t in range(_BLK // _TILE):
            indep_tile(f2_ref, adj2_ref, out2_ref, t)
            indep_tile(f1_ref, adj1_ref, out1_ref, t)

    @pl.when(pid == 0)
    def _():
        for t in range(_BLK // _TILE):
            indep_tile(f2_ref, adj2_ref, out2_ref, t)
        f = f1_ref[:]
        xw = jnp.dot(f, w1_ref[:], preferred_element_type=jnp.float32)
        nrm = jnp.sqrt(jnp.sum(f * f, axis=1, keepdims=True))
        gram = jax.lax.dot_general(
            f, f, (((1,), (1,)), ((), ())),
            preferred_element_type=jnp.float32,
            precision=jax.lax.Precision.HIGHEST)
        denom = jnp.maximum(nrm * jnp.transpose(nrm), 1e-8)
        sim = gram / denom
        mn = jnp.min(sim)
        mx = jnp.max(sim)
        simn = (sim - mn) / (mx - mn)
        deg = jnp.sum(simn, axis=1, keepdims=True)
        dinv = jnp.where(deg > 0, deg ** -0.5, 0.0)
        p = dinv * xw
        q = jnp.dot(simn, p, preferred_element_type=jnp.float32,
                    precision=jax.lax.Precision.HIGHEST)
        conv = dinv * q

        x = jnp.maximum(conv + b1_ref[:], 0.0)
        x = jnp.maximum(x * _pool_scale(adj1_ref[:]), 0.0)

        h = x
        for i in range(2):
            m = jnp.dot(h, wg_ref[i], preferred_element_type=jnp.float32)
            aggvec = jnp.sum(m, axis=0, keepdims=True)
            gi = jnp.dot(aggvec, wih_ref[:],
                         preferred_element_type=jnp.float32) + bih_ref[:]
            gh = jnp.dot(h, whh_ref[:],
                         preferred_element_type=jnp.float32) + bhh_ref[:]
            h = _gru(h, gi, gh)

        out1_ref[:] = jnp.maximum(h, 0.0)


@jax.jit
def kernel(input1, input2, adj_sem_ori, adj_sem_gcn, W1, b1, Wg, W_ih, W_hh,
           b_ih, b_hh):
    b, s, d = input1.shape
    blk = pl.BlockSpec((_BLK, _D), lambda i: (i, 0))
    const2 = lambda i: (0, 0)
    out1, out2 = pl.pallas_call(
        _gcn_body,
        grid=(_HALF // _BLK,),
        in_specs=[
            blk, blk, blk, blk,
            pl.BlockSpec((_D, _H), const2),
            pl.BlockSpec((1, _H), const2),
            pl.BlockSpec((2, _H, _H), lambda i: (0, 0, 0)),
            pl.BlockSpec((_H, 3 * _H), const2),
            pl.BlockSpec((_H, 3 * _H), const2),
            pl.BlockSpec((1, 3 * _H), const2),
            pl.BlockSpec((1, 3 * _H), const2),
        ],
        out_specs=[pl.BlockSpec((_BLK, _H), lambda i: (i, 0)),
                   pl.BlockSpec((_BLK, _H), lambda i: (i, 0))],
        out_shape=[jax.ShapeDtypeStruct((_HALF, _H), jnp.float32),
                   jax.ShapeDtypeStruct((_HALF, _H), jnp.float32)],
        compiler_params=pltpu.CompilerParams(
            dimension_semantics=("arbitrary",)),
    )(input1.reshape(-1, d), adj_sem_ori.reshape(-1, s),
      input2.reshape(-1, d), adj_sem_gcn.reshape(-1, s),
      W1, b1.reshape(1, -1), Wg, W_ih.T, W_hh.T,
      b_ih.reshape(1, -1), b_hh.reshape(1, -1))
    return (out1.reshape(b, s, _H), out2.reshape(b, s, _H))


# speedup vs baseline: 1.2377x; 1.2377x over previous
import jax
import jax.numpy as jnp
from jax.experimental import pallas as pl
from jax.experimental.pallas import tpu as pltpu

_D = 256
_H = 256
_BLK = 1024
_TILE = 256
_HALF = 4096


def _gru(h, gi, gh):
    r = jax.nn.sigmoid(gi[:, :_H] + gh[:, :_H])
    z = jax.nn.sigmoid(gi[:, _H:2 * _H] + gh[:, _H:2 * _H])
    n = jnp.tanh(gi[:, 2 * _H:] + r * gh[:, 2 * _H:])
    return (1.0 - z) * n + z * h


def _pool_scale(adj):
    smean = jnp.mean(adj, axis=1, keepdims=True)
    smax = jnp.max(adj, axis=1, keepdims=True)
    return 1.0 + smean + smax


def _gcn_body(f1_ref, adj1_ref, f2_ref, adj2_ref, w1_ref, b1_ref, wg_ref,
              wih_ref, whh_ref, bih_ref, bhh_ref, out1_ref, out2_ref):
    pid = pl.program_id(0)

    def indep_tile(f_ref, adj_ref, out_ref, t):
        rs = pl.ds(t * _TILE, _TILE)
        f = f_ref[rs, :]
        xw = jnp.dot(f, w1_ref[:], preferred_element_type=jnp.float32)
        x = jnp.maximum(xw + b1_ref[:], 0.0)
        x = jnp.maximum(x * _pool_scale(adj_ref[rs, :]), 0.0)
        bsum = bih_ref[:] + bhh_ref[:]
        h = x
        for _ in range(2):
            g = jnp.dot(h, whh_ref[:], preferred_element_type=jnp.float32)
            r = jax.nn.sigmoid(g[:, :_H] + bsum[:, :_H])
            z = jax.nn.sigmoid(g[:, _H:2 * _H] + bsum[:, _H:2 * _H])
            hn = g[:, 2 * _H:] + bhh_ref[:, 2 * _H:]
            n = jnp.tanh(bih_ref[:, 2 * _H:] + r * hn)
            h = (1.0 - z) * n + z * h
        out_ref[rs, :] = jnp.maximum(h, 0.0)

    @pl.when(pid != 0)
    def _():
        for t in range(_BLK // _TILE):
            indep_tile(f2_ref, adj2_ref, out2_ref, t)
            indep_tile(f1_ref, adj1_ref, out1_ref, t)

    @pl.when(pid == 0)
    def _():
        for t in range(_BLK // _TILE):
            indep_tile(f2_ref, adj2_ref, out2_ref, t)
            if t * _TILE >= 512:
                indep_tile(f1_ref, adj1_ref, out1_ref, t)
        f = f1_ref[pl.ds(0, 512), :]
        xw = jnp.dot(f, w1_ref[:], preferred_element_type=jnp.float32)
        nrm = jnp.sqrt(jnp.sum(f * f, axis=1, keepdims=True))
        gram = jax.lax.dot_general(
            f, f, (((1,), (1,)), ((), ())),
            preferred_element_type=jnp.float32,
            precision=jax.lax.Precision.HIGHEST)
        denom = jnp.maximum(nrm * jnp.transpose(nrm), 1e-8)
        sim = gram / denom
        mn = jnp.min(sim)
        mx = jnp.max(sim)
        simn = (sim - mn) / (mx - mn)
        deg = jnp.sum(simn, axis=1, keepdims=True)
        dinv = jnp.where(deg > 0, deg ** -0.5, 0.0)
        p = dinv * xw
        q = jnp.dot(simn, p, preferred_element_type=jnp.float32,
                    precision=jax.lax.Precision.HIGHEST)
        conv = dinv * q

        x = jnp.maximum(conv + b1_ref[:], 0.0)
        x = jnp.maximum(x * _pool_scale(adj1_ref[pl.ds(0, 512), :]), 0.0)

        h = x
        for i in range(2):
            m = jnp.dot(h, wg_ref[i], preferred_element_type=jnp.float32)
            aggvec = jnp.sum(m, axis=0, keepdims=True)
            gi = jnp.dot(aggvec, wih_ref[:],
                         preferred_element_type=jnp.float32) + bih_ref[:]
            gh = jnp.dot(h, whh_ref[:],
                         preferred_element_type=jnp.float32) + bhh_ref[:]
            h = _gru(h, gi, gh)

        out1_ref[pl.ds(0, 512), :] = jnp.maximum(h, 0.0)


@jax.jit
def kernel(input1, input2, adj_sem_ori, adj_sem_gcn, W1, b1, Wg, W_ih, W_hh,
           b_ih, b_hh):
    b, s, d = input1.shape
    blk = pl.BlockSpec((_BLK, _D), lambda i: (i, 0))
    const2 = lambda i: (0, 0)
    out1, out2 = pl.pallas_call(
        _gcn_body,
        grid=(_HALF // _BLK,),
        in_specs=[
            blk, blk, blk, blk,
            pl.BlockSpec((_D, _H), const2),
            pl.BlockSpec((1, _H), const2),
            pl.BlockSpec((2, _H, _H), lambda i: (0, 0, 0)),
            pl.BlockSpec((_H, 3 * _H), const2),
            pl.BlockSpec((_H, 3 * _H), const2),
            pl.BlockSpec((1, 3 * _H), const2),
            pl.BlockSpec((1, 3 * _H), const2),
        ],
        out_specs=[pl.BlockSpec((_BLK, _H), lambda i: (i, 0)),
                   pl.BlockSpec((_BLK, _H), lambda i: (i, 0))],
        out_shape=[jax.ShapeDtypeStruct((_HALF, _H), jnp.float32),
                   jax.ShapeDtypeStruct((_HALF, _H), jnp.float32)],
        compiler_params=pltpu.CompilerParams(
            dimension_semantics=("arbitrary",)),
    )(input1.reshape(-1, d), adj_sem_ori.reshape(-1, s),
      input2.reshape(-1, d), adj_sem_gcn.reshape(-1, s),
      W1, b1.reshape(1, -1), Wg, W_ih.T, W_hh.T,
      b_ih.reshape(1, -1), b_hh.reshape(1, -1))
    return (out1.reshape(b, s, _H), out2.reshape(b, s, _H))


# speedup vs baseline: 1.2378x; 1.0001x over previous
import jax
import jax.numpy as jnp
from jax.experimental import pallas as pl
from jax.experimental.pallas import tpu as pltpu

_D = 256
_H = 256
_BLK = 512
_TILE = 256
_HALF = 4096


def _gru(h, gi, gh):
    r = jax.nn.sigmoid(gi[:, :_H] + gh[:, :_H])
    z = jax.nn.sigmoid(gi[:, _H:2 * _H] + gh[:, _H:2 * _H])
    n = jnp.tanh(gi[:, 2 * _H:] + r * gh[:, 2 * _H:])
    return (1.0 - z) * n + z * h


def _pool_scale(adj):
    smean = jnp.mean(adj, axis=1, keepdims=True)
    smax = jnp.max(adj, axis=1, keepdims=True)
    return 1.0 + smean + smax


def _gcn_body(f1_ref, adj1_ref, f2_ref, adj2_ref, w1_ref, b1_ref, wg_ref,
              wih_ref, whh_ref, bih_ref, bhh_ref, out1_ref, out2_ref):
    pid = pl.program_id(0)

    def indep_tile(f_ref, adj_ref, out_ref, t):
        rs = pl.ds(t * _TILE, _TILE)
        f = f_ref[rs, :]
        xw = jnp.dot(f, w1_ref[:], preferred_element_type=jnp.float32)
        x = jnp.maximum(xw + b1_ref[:], 0.0)
        x = jnp.maximum(x * _pool_scale(adj_ref[rs, :]), 0.0)
        bsum = bih_ref[:] + bhh_ref[:]
        h = x
        for _ in range(2):
            g = jnp.dot(h, whh_ref[:], preferred_element_type=jnp.float32)
            r = jax.nn.sigmoid(g[:, :_H] + bsum[:, :_H])
            z = jax.nn.sigmoid(g[:, _H:2 * _H] + bsum[:, _H:2 * _H])
            hn = g[:, 2 * _H:] + bhh_ref[:, 2 * _H:]
            n = jnp.tanh(bih_ref[:, 2 * _H:] + r * hn)
            h = (1.0 - z) * n + z * h
        out_ref[rs, :] = jnp.maximum(h, 0.0)

    @pl.when(pid != 0)
    def _():
        for t in range(_BLK // _TILE):
            indep_tile(f2_ref, adj2_ref, out2_ref, t)
            indep_tile(f1_ref, adj1_ref, out1_ref, t)

    @pl.when(pid == 0)
    def _():
        for t in range(_BLK // _TILE):
            indep_tile(f2_ref, adj2_ref, out2_ref, t)
        f = f1_ref[:]
        xw = jnp.dot(f, w1_ref[:], preferred_element_type=jnp.float32)
        nrm = jnp.sqrt(jnp.sum(f * f, axis=1, keepdims=True))
        gram = jax.lax.dot_general(
            f, f, (((1,), (1,)), ((), ())),
            preferred_element_type=jnp.float32,
            precision=jax.lax.Precision.HIGHEST)
        denom = jnp.maximum(nrm * jnp.transpose(nrm), 1e-8)
        sim = gram / denom
        mn = jnp.min(sim)
        mx = jnp.max(sim)
        simn = (sim - mn) / (mx - mn)
        deg = jnp.sum(simn, axis=1, keepdims=True)
        dinv = jnp.where(deg > 0, deg ** -0.5, 0.0)
        p = dinv * xw
        q = jnp.dot(simn, p, preferred_element_type=jnp.float32,
                    precision=jax.lax.Precision.HIGHEST)
        conv = dinv * q

        x = jnp.maximum(conv + b1_ref[:], 0.0)
        x = jnp.maximum(x * _pool_scale(adj1_ref[:]), 0.0)

        h = x
        for i in range(2):
            m = jnp.dot(h, wg_ref[i], preferred_element_type=jnp.float32)
            aggvec = jnp.sum(m, axis=0, keepdims=True)
            gi = jnp.dot(aggvec, wih_ref[:],
                         preferred_element_type=jnp.float32) + bih_ref[:]
            gh = jnp.dot(h, whh_ref[:],
                         preferred_element_type=jnp.float32) + bhh_ref[:]
            h = _gru(h, gi, gh)

        out1_ref[:] = jnp.maximum(h, 0.0)


@jax.jit
def kernel(input1, input2, adj_sem_ori, adj_sem_gcn, W1, b1, Wg, W_ih, W_hh,
           b_ih, b_hh):
    b, s, d = input1.shape
    blk = pl.BlockSpec((_BLK, _D), lambda i: (i, 0))
    const2 = lambda i: (0, 0)
    out1, out2 = pl.pallas_call(
        _gcn_body,
        grid=(_HALF // _BLK,),
        in_specs=[
            blk, blk, blk, blk,
            pl.BlockSpec((_D, _H), const2),
            pl.BlockSpec((1, _H), const2),
            pl.BlockSpec((2, _H, _H), lambda i: (0, 0, 0)),
            pl.BlockSpec((_H, 3 * _H), const2),
            pl.BlockSpec((_H, 3 * _H), const2),
            pl.BlockSpec((1, 3 * _H), const2),
            pl.BlockSpec((1, 3 * _H), const2),
        ],
        out_specs=[pl.BlockSpec((_BLK, _H), lambda i: (i, 0)),
                   pl.BlockSpec((_BLK, _H), lambda i: (i, 0))],
        out_shape=[jax.ShapeDtypeStruct((_HALF, _H), jnp.float32),
                   jax.ShapeDtypeStruct((_HALF, _H), jnp.float32)],
        compiler_params=pltpu.CompilerParams(
            dimension_semantics=("arbitrary",)),
    )(input1.reshape(-1, d), adj_sem_ori.reshape(-1, s),
      input2.reshape(-1, d), adj_sem_gcn.reshape(-1, s),
      W1, b1.reshape(1, -1), Wg, W_ih.T, W_hh.T,
      b_ih.reshape(1, -1), b_hh.reshape(1, -1))
    return (out1.reshape(b, s, _H), out2.reshape(b, s, _H))
